# speedup vs baseline: 1.4356x; 1.4356x over previous
import functools

import jax
import jax.numpy as jnp
from jax import lax
from jax.experimental import pallas as pl
from jax.experimental.pallas import tpu as pltpu
from jax.experimental.pallas import tpu_sc as plsc

LAMB = 0.7
N = 10000
D = 128
LANES = 16
NC, NS = 2, 16
NW = NC * NS
BPW = 312
TAIL = N - NW * BPW
CHUNK = 104


def _mix_body(x_hbm, idx_hbm, out_hbm, idx_v, a_v, b_v, sem):
    wid = lax.axis_index("s") * NC + lax.axis_index("c")

    def do_block(base, nrows, chunk):
        pltpu.sync_copy(idx_hbm.at[pl.ds(base, nrows)],
                        idx_v.at[pl.ds(0, nrows)])
        cps = [pltpu.async_copy(x_hbm.at[pl.ds(base, nrows)],
                                a_v.at[pl.ds(0, nrows)], sem)]
        for c in range(0, nrows, chunk):
            w = min(chunk, nrows - c)
            cps.append(pltpu.async_copy(x_hbm.at[idx_v.at[pl.ds(c, w)]],
                                        b_v.at[pl.ds(c, w)],
                                        sem))
        for cp in cps:
            cp.wait()

        def row(r, _):
            for cc in range(D // LANES):
                sl = pl.ds(cc * LANES, LANES)
                a = a_v[r, sl]
                b = b_v[r, sl]
                a_v[r, sl] = a * LAMB + b * (1.0 - LAMB)
            return _

        lax.fori_loop(0, nrows, row, 0)
        pltpu.sync_copy(a_v.at[pl.ds(0, nrows)],
                        out_hbm.at[pl.ds(base, nrows)])

    do_block(wid * BPW, BPW, CHUNK)

    @pl.when(wid == NW - 1)
    def _():
        do_block(NW * BPW, TAIL, TAIL)


@functools.lru_cache(maxsize=1)
def _build():
    mesh = plsc.VectorSubcoreMesh(core_axis_name="c", subcore_axis_name="s",
                                  num_cores=NC, num_subcores=NS)
    return pl.kernel(
        _mix_body,
        out_type=jax.ShapeDtypeStruct((N, D), jnp.float32),
        mesh=mesh,
        scratch_types=[
            pltpu.VMEM((BPW,), jnp.int32),
            pltpu.VMEM((BPW, D), jnp.float32),
            pltpu.VMEM((BPW, D), jnp.float32),
            pltpu.SemaphoreType.DMA,
        ],
    )


def kernel(x, y, edge_index, pair_idx):
    x_mix = _build()(x, pair_idx)
    return (x_mix, y, edge_index)


# speedup vs baseline: 1.5461x; 1.0770x over previous
import functools

import jax
import jax.numpy as jnp
from jax import lax
from jax.experimental import pallas as pl
from jax.experimental.pallas import tpu as pltpu
from jax.experimental.pallas import tpu_sc as plsc

LAMB = 0.7
N = 10000
D = 128
E = 320000
LANES = 16
NC, NS = 2, 16
NW = NC * NS
ROWS = 320
SPAN = N - ROWS
HALF = ROWS // 2
GCH = 80
ECH = 10240
ESPAN = E - ECH


def _mix_body(x_hbm, idx_hbm, y_hbm, edge_hbm,
              out_hbm, y_out, edge_out,
              idx_v, a_v, b_v, y_v, e_v, sem0, sem1, sem_st, sem_ps):
    wid = lax.axis_index("s") * NC + lax.axis_index("c")
    base = (wid * SPAN // (NW - 1)) // 8 * 8
    ebase = (wid * ESPAN // (NW - 1)) // 128 * 128

    ps_in = [pltpu.async_copy(y_hbm.at[pl.ds(base, ROWS)], y_v, sem_ps),
             pltpu.async_copy(edge_hbm.at[:, pl.ds(ebase, ECH)], e_v, sem_ps)]

    pltpu.sync_copy(idx_hbm.at[pl.ds(base, ROWS)], idx_v)

    cps = []
    for h, sem in ((0, sem0), (1, sem1)):
        r0 = h * HALF
        half = [pltpu.async_copy(x_hbm.at[pl.ds(base + r0, HALF)],
                                 a_v.at[pl.ds(r0, HALF)], sem)]
        for c in range(r0, r0 + HALF, GCH):
            half.append(pltpu.async_copy(x_hbm.at[idx_v.at[pl.ds(c, GCH)]],
                                         b_v.at[pl.ds(c, GCH)], sem))
        cps.append(half)

    def blend_rows(r0):
        def row(r, _):
            for cc in range(D // LANES):
                sl = pl.ds(cc * LANES, LANES)
                a_v[r, sl] = a_v[r, sl] * LAMB + b_v[r, sl] * (1.0 - LAMB)
            return _
        lax.fori_loop(r0, r0 + HALF, row, 0)

    sts = []
    for h in (0, 1):
        for cp in cps[h]:
            cp.wait()
        blend_rows(h * HALF)
        sts.append(pltpu.async_copy(a_v.at[pl.ds(h * HALF, HALF)],
                                    out_hbm.at[pl.ds(base + h * HALF, HALF)],
                                    sem_st))
    for cp in ps_in:
        cp.wait()
    sts.append(pltpu.async_copy(y_v, y_out.at[pl.ds(base, ROWS)], sem_st))
    sts.append(pltpu.async_copy(e_v, edge_out.at[:, pl.ds(ebase, ECH)], sem_st))
    for cp in sts:
        cp.wait()


@functools.lru_cache(maxsize=1)
def _build():
    mesh = plsc.VectorSubcoreMesh(core_axis_name="c", subcore_axis_name="s",
                                  num_cores=NC, num_subcores=NS)
    return pl.kernel(
        _mix_body,
        out_type=(jax.ShapeDtypeStruct((N, D), jnp.float32),
                  jax.ShapeDtypeStruct((N,), jnp.int32),
                  jax.ShapeDtypeStruct((2, E), jnp.int32)),
        mesh=mesh,
        scratch_types=[
            pltpu.VMEM((ROWS,), jnp.int32),
            pltpu.VMEM((ROWS, D), jnp.float32),
            pltpu.VMEM((ROWS, D), jnp.float32),
            pltpu.VMEM((ROWS,), jnp.int32),
            pltpu.VMEM((2, ECH), jnp.int32),
            pltpu.SemaphoreType.DMA,
            pltpu.SemaphoreType.DMA,
            pltpu.SemaphoreType.DMA,
            pltpu.SemaphoreType.DMA,
        ],
    )


def kernel(x, y, edge_index, pair_idx):
    x_mix, new_y, edge_out = _build()(x, pair_idx, y, edge_index)
    return (x_mix, new_y, edge_out)


# speedup vs baseline: 1.5637x; 1.0114x over previous
import functools

import jax
import jax.numpy as jnp
from jax import lax
from jax.experimental import pallas as pl
from jax.experimental.pallas import tpu as pltpu
from jax.experimental.pallas import tpu_sc as plsc

LAMB = 0.7
N = 10000
D = 128
E = 320000
LANES = 16
NC, NS = 2, 16
NW = NC * NS
ROWS = 320
SPAN = N - ROWS
HALF = ROWS // 2
GCH = 80
ECH = 10240
ESPAN = E - ECH


def _mix_body(x_hbm, idx_hbm, y_hbm, edge_hbm,
              out_hbm, y_out, edge_out,
              idx_v, a_v, b_v, y_v, e_v,
              sem_c0, sem_c1, sem_c2, sem_c3, sem_st, sem_ps):
    wid = lax.axis_index("s") * NC + lax.axis_index("c")
    base = (wid * SPAN // (NW - 1)) // 8 * 8
    ebase = (wid * ESPAN // (NW - 1)) // 128 * 128
    sems = (sem_c0, sem_c1, sem_c2, sem_c3)
    nch = ROWS // GCH

    cps = [[pltpu.async_copy(x_hbm.at[pl.ds(base, GCH)],
                             a_v.at[pl.ds(0, GCH)], sems[0])]]
    pltpu.sync_copy(idx_hbm.at[pl.ds(base, ROWS)], idx_v)
    cps[0].append(pltpu.async_copy(x_hbm.at[idx_v.at[pl.ds(0, GCH)]],
                                   b_v.at[pl.ds(0, GCH)], sems[0]))
    for c in range(1, nch):
        r0 = c * GCH
        cps.append([
            pltpu.async_copy(x_hbm.at[pl.ds(base + r0, GCH)],
                             a_v.at[pl.ds(r0, GCH)], sems[c]),
            pltpu.async_copy(x_hbm.at[idx_v.at[pl.ds(r0, GCH)]],
                             b_v.at[pl.ds(r0, GCH)], sems[c])])
    ps_in = [pltpu.async_copy(y_hbm.at[pl.ds(base, ROWS)], y_v, sem_ps),
             pltpu.async_copy(edge_hbm.at[:, pl.ds(ebase, ECH)], e_v, sem_ps)]

    def blend_rows(r0):
        def rows2(i, _):
            r = r0 + i * 2
            for dr in range(2):
                for cc in range(D // LANES):
                    sl = pl.ds(cc * LANES, LANES)
                    a_v[r + dr, sl] = (a_v[r + dr, sl] * LAMB
                                       + b_v[r + dr, sl] * (1.0 - LAMB))
            return _
        lax.fori_loop(0, GCH // 2, rows2, 0)

    sts = []
    for c in range(nch):
        for cp in cps[c]:
            cp.wait()
        blend_rows(c * GCH)
        sts.append(pltpu.async_copy(a_v.at[pl.ds(c * GCH, GCH)],
                                    out_hbm.at[pl.ds(base + c * GCH, GCH)],
                                    sem_st))
    for cp in ps_in:
        cp.wait()
    sts.append(pltpu.async_copy(y_v, y_out.at[pl.ds(base, ROWS)], sem_st))
    sts.append(pltpu.async_copy(e_v, edge_out.at[:, pl.ds(ebase, ECH)], sem_st))
    for cp in sts:
        cp.wait()


@functools.lru_cache(maxsize=1)
def _build():
    mesh = plsc.VectorSubcoreMesh(core_axis_name="c", subcore_axis_name="s",
                                  num_cores=NC, num_subcores=NS)
    return pl.kernel(
        _mix_body,
        out_type=(jax.ShapeDtypeStruct((N, D), jnp.float32),
                  jax.ShapeDtypeStruct((N,), jnp.int32),
                  jax.ShapeDtypeStruct((2, E), jnp.int32)),
        mesh=mesh,
        scratch_types=[
            pltpu.VMEM((ROWS,), jnp.int32),
            pltpu.VMEM((ROWS, D), jnp.float32),
            pltpu.VMEM((ROWS, D), jnp.float32),
            pltpu.VMEM((ROWS,), jnp.int32),
            pltpu.VMEM((2, ECH), jnp.int32),
            pltpu.SemaphoreType.DMA,
            pltpu.SemaphoreType.DMA,
            pltpu.SemaphoreType.DMA,
            pltpu.SemaphoreType.DMA,
            pltpu.SemaphoreType.DMA,
            pltpu.SemaphoreType.DMA,
        ],
    )


def kernel(x, y, edge_index, pair_idx):
    x_mix, new_y, edge_out = _build()(x, pair_idx, y, edge_index)
    return (x_mix, new_y, edge_out)


# speedup vs baseline: 1.6256x; 1.0396x over previous
import functools

import jax
import jax.numpy as jnp
from jax import lax
from jax.experimental import pallas as pl
from jax.experimental.pallas import tpu as pltpu
from jax.experimental.pallas import tpu_sc as plsc

LAMB = 0.7
N = 10000
D = 128
E = 320000
LANES = 16
NC, NS = 2, 16
NW = NC * NS
ROWS = 320
SPAN = N - ROWS
GCH = 80


def _mix_body(x_hbm, idx_hbm, out_hbm,
              idx_v, a_v, b_v, sem_c0, sem_c1, sem_c2, sem_c3, sem_st):
    wid = lax.axis_index("s") * NC + lax.axis_index("c")
    base = (wid * SPAN // (NW - 1)) // 8 * 8
    sems = (sem_c0, sem_c1, sem_c2, sem_c3)
    nch = ROWS // GCH

    cps = [[pltpu.async_copy(x_hbm.at[pl.ds(base, GCH)],
                             a_v.at[pl.ds(0, GCH)], sems[0])]]
    pltpu.sync_copy(idx_hbm.at[pl.ds(base, ROWS)], idx_v)
    cps[0].append(pltpu.async_copy(x_hbm.at[idx_v.at[pl.ds(0, GCH)]],
                                   b_v.at[pl.ds(0, GCH)], sems[0]))
    for c in range(1, nch):
        r0 = c * GCH
        cps.append([
            pltpu.async_copy(x_hbm.at[pl.ds(base + r0, GCH)],
                             a_v.at[pl.ds(r0, GCH)], sems[c]),
            pltpu.async_copy(x_hbm.at[idx_v.at[pl.ds(r0, GCH)]],
                             b_v.at[pl.ds(r0, GCH)], sems[c])])

    def blend_rows(r0):
        def rows2(i, _):
            r = r0 + i * 2
            for dr in range(2):
                for cc in range(D // LANES):
                    sl = pl.ds(cc * LANES, LANES)
                    a_v[r + dr, sl] = (a_v[r + dr, sl] * LAMB
                                       + b_v[r + dr, sl] * (1.0 - LAMB))
            return _
        lax.fori_loop(0, GCH // 2, rows2, 0)

    sts = []
    for c in range(nch):
        for cp in cps[c]:
            cp.wait()
        blend_rows(c * GCH)
        sts.append(pltpu.async_copy(a_v.at[pl.ds(c * GCH, GCH)],
                                    out_hbm.at[pl.ds(base + c * GCH, GCH)],
                                    sem_st))
    for cp in sts:
        cp.wait()


def _copy_body(y_ref, e_ref, y_out, e_out):
    y_out[...] = y_ref[...]
    e_out[...] = e_ref[...]


@functools.lru_cache(maxsize=1)
def _build():
    mesh = plsc.VectorSubcoreMesh(core_axis_name="c", subcore_axis_name="s",
                                  num_cores=NC, num_subcores=NS)
    return pl.kernel(
        _mix_body,
        out_type=jax.ShapeDtypeStruct((N, D), jnp.float32),
        mesh=mesh,
        scratch_types=[
            pltpu.VMEM((ROWS,), jnp.int32),
            pltpu.VMEM((ROWS, D), jnp.float32),
            pltpu.VMEM((ROWS, D), jnp.float32),
            pltpu.SemaphoreType.DMA,
            pltpu.SemaphoreType.DMA,
            pltpu.SemaphoreType.DMA,
            pltpu.SemaphoreType.DMA,
            pltpu.SemaphoreType.DMA,
        ],
    )


@functools.lru_cache(maxsize=1)
def _build_copy():
    return pl.pallas_call(
        _copy_body,
        out_shape=(jax.ShapeDtypeStruct((N,), jnp.int32),
                   jax.ShapeDtypeStruct((2, E), jnp.int32)),
    )


def kernel(x, y, edge_index, pair_idx):
    x_mix = _build()(x, pair_idx)
    new_y, edge_out = _build_copy()(y, edge_index)
    return (x_mix, new_y, edge_out)
